# baseline (speedup 1.0000x reference)
import functools
import math

import jax
import jax.numpy as jnp
from jax import lax
from jax.experimental import pallas as pl
from jax.experimental.pallas import tpu as pltpu
from jax.experimental.pallas import tpu_sc as plsc

_WN = 1000
_TN = 5000
_L = 4
_A = 128
_K = _TN * _L
_LANES = 16
_KCH = _K // _LANES
_NTILES = 32
_RPT = 32
_LN3 = math.log(3.0)


def _sc_body(wf_hbm, par_hbm, tau_hbm, out_hbm,
             wf_v, par_v, tau_v, row0, row1, ac_v,
             sem_in, sem0, sem1):
    cid = lax.axis_index("c")
    sid = lax.axis_index("s")
    wid = sid * 2 + cid
    base = wid * _RPT

    pltpu.sync_copy(par_hbm, par_v)
    pltpu.sync_copy(tau_hbm, tau_v)
    pltpu.sync_copy(wf_hbm.at[wid], wf_v)

    bvec = par_v[pl.ds(_A, _LANES)]

    for h in range(_RPT // _LANES):
        zvec = bvec
        for ch in range(_A // _LANES):
            pv = par_v[pl.ds(ch * _LANES, _LANES)]
            for j in range(_LANES):
                f = ch * _LANES + j
                zvec = zvec + wf_v[pl.ds(f * _RPT + h * _LANES, _LANES)] * pv[j]
        ac_v[pl.ds(h * _LANES, _LANES)] = zvec + _LN3
        ac_v[pl.ds(_RPT + h * _LANES, _LANES)] = 1.0 / (3.0 * (1.0 + jnp.exp(zvec)))

    bufs = (row0, row1)
    sems = (sem0, sem1)
    handles = [None, None]
    for w in range(_RPT):
        slot = w % 2
        buf = bufs[slot]
        if handles[slot] is not None:
            handles[slot].wait()
        h, ln = w // _LANES, w % _LANES
        aw = ac_v[pl.ds(h * _LANES, _LANES)][ln]
        cw = ac_v[pl.ds(_RPT + h * _LANES, _LANES)][ln]

        def chunk(k, _, buf=buf, aw=aw, cw=cw):
            o = k * _LANES
            buf[pl.ds(o, _LANES)] = cw * jnp.exp(tau_v[pl.ds(o, _LANES)] * aw)
            return _

        lax.fori_loop(0, _KCH, chunk, None, unroll=4)
        idx = jnp.minimum(base + w, _WN - 1)
        handles[slot] = pltpu.async_copy(buf, out_hbm.at[idx], sems[slot])
    handles[0].wait()
    handles[1].wait()


@jax.jit
def _run(wf, par, tau):
    mesh = plsc.VectorSubcoreMesh(core_axis_name="c", subcore_axis_name="s")
    f = functools.partial(
        pl.kernel,
        mesh=mesh,
        out_type=jax.ShapeDtypeStruct((_WN, _K), jnp.float32),
        scratch_types=[
            pltpu.VMEM((_A * _RPT,), jnp.float32),
            pltpu.VMEM((_A + _LANES,), jnp.float32),
            pltpu.VMEM((_K,), jnp.float32),
            pltpu.VMEM((_K,), jnp.float32),
            pltpu.VMEM((_K,), jnp.float32),
            pltpu.VMEM((2 * _RPT,), jnp.float32),
            pltpu.SemaphoreType.DMA,
            pltpu.SemaphoreType.DMA,
            pltpu.SemaphoreType.DMA,
        ],
    )(_sc_body)
    return f(wf, par, tau)


def kernel(inputs, W, b):
    wf = inputs[:_WN, :_A]
    pad = jnp.broadcast_to(wf[_WN - 1], (_NTILES * _RPT - _WN, _A))
    wf = jnp.concatenate([wf, pad])
    wf = wf.reshape(_NTILES, _RPT, _A).transpose(0, 2, 1).reshape(_NTILES, _A * _RPT)
    tau = inputs[_WN:, :_L].reshape(_K)
    par = jnp.concatenate([W[:, 0], jnp.broadcast_to(b, (_LANES,))])
    out = _run(wf, par, tau)
    return out.reshape(_WN, _TN, _L)


# speedup vs baseline: 1.8817x; 1.8817x over previous
import functools
import math

import jax
import jax.numpy as jnp
from jax import lax
from jax.experimental import pallas as pl
from jax.experimental.pallas import tpu as pltpu
from jax.experimental.pallas import tpu_sc as plsc

_WN = 1000
_TN = 5000
_L = 4
_A = 128
_K = _TN * _L
_LANES = 16
_KCH = _K // _LANES
_NTILES = 32
_RPT = 32
_LN3 = math.log(3.0)


def _sc_body(wf_hbm, par_hbm, tau_hbm, out_hbm,
             wf_v, par_v, tau_v, row0, row1, ac_v,
             sem_in, sem0, sem1):
    cid = lax.axis_index("c")
    sid = lax.axis_index("s")
    wid = sid * 2 + cid
    base = wid * _RPT

    pltpu.sync_copy(par_hbm, par_v)
    pltpu.sync_copy(tau_hbm, tau_v)
    pltpu.sync_copy(wf_hbm.at[wid], wf_v)

    bvec = par_v[pl.ds(_A, _LANES)]

    for h in range(_RPT // _LANES):
        zvec = bvec
        for ch in range(_A // _LANES):
            pv = par_v[pl.ds(ch * _LANES, _LANES)]
            for j in range(_LANES):
                f = ch * _LANES + j
                zvec = zvec + wf_v[pl.ds(f * _RPT + h * _LANES, _LANES)] * pv[j]
        ac_v[pl.ds(h * _LANES, _LANES)] = zvec + _LN3
        ac_v[pl.ds(_RPT + h * _LANES, _LANES)] = 1.0 / (3.0 * (1.0 + jnp.exp(zvec)))

    bufs = (row0, row1)
    sems = (sem0, sem1)
    handles = [None, None]
    for w in range(_RPT):
        slot = w % 2
        buf = bufs[slot]
        if handles[slot] is not None:
            handles[slot].wait()
        h, ln = w // _LANES, w % _LANES
        aw = ac_v[pl.ds(h * _LANES, _LANES)][ln]
        cw = ac_v[pl.ds(_RPT + h * _LANES, _LANES)][ln]

        @plsc.parallel_loop(0, _K, _LANES, unroll=8)
        def _row(o, buf=buf, aw=aw, cw=cw):
            buf[pl.ds(o, _LANES)] = cw * jnp.exp(tau_v[pl.ds(o, _LANES)] * aw)
        idx = jnp.minimum(base + w, _WN - 1)
        handles[slot] = pltpu.async_copy(buf, out_hbm.at[idx], sems[slot])
    handles[0].wait()
    handles[1].wait()


@jax.jit
def _run(wf, par, tau):
    mesh = plsc.VectorSubcoreMesh(core_axis_name="c", subcore_axis_name="s")
    f = functools.partial(
        pl.kernel,
        mesh=mesh,
        out_type=jax.ShapeDtypeStruct((_WN, _K), jnp.float32),
        scratch_types=[
            pltpu.VMEM((_A * _RPT,), jnp.float32),
            pltpu.VMEM((_A + _LANES,), jnp.float32),
            pltpu.VMEM((_K,), jnp.float32),
            pltpu.VMEM((_K,), jnp.float32),
            pltpu.VMEM((_K,), jnp.float32),
            pltpu.VMEM((2 * _RPT,), jnp.float32),
            pltpu.SemaphoreType.DMA,
            pltpu.SemaphoreType.DMA,
            pltpu.SemaphoreType.DMA,
        ],
    )(_sc_body)
    return f(wf, par, tau)


def kernel(inputs, W, b):
    wf = inputs[:_WN, :_A]
    pad = jnp.broadcast_to(wf[_WN - 1], (_NTILES * _RPT - _WN, _A))
    wf = jnp.concatenate([wf, pad])
    wf = wf.reshape(_NTILES, _RPT, _A).transpose(0, 2, 1).reshape(_NTILES, _A * _RPT)
    tau = inputs[_WN:, :_L].reshape(_K)
    par = jnp.concatenate([W[:, 0], jnp.broadcast_to(b, (_LANES,))])
    out = _run(wf, par, tau)
    return out.reshape(_WN, _TN, _L)
